# Initial kernel scaffold; baseline (speedup 1.0000x reference)
import jax
import jax.numpy as jnp
from jax.experimental import pallas as pl


def kernel(x, spatial_table, temporal_table, mask_table, W, b):
    raise NotImplementedError("write your pallas kernel here")


# speedup vs baseline: 1.2034x; 1.2034x over previous
import functools

import jax
import jax.numpy as jnp
import numpy as np
from jax import lax
from jax.experimental import pallas as pl
from jax.experimental.pallas import tpu as pltpu
from jax.experimental.pallas import tpu_sc as plsc

N_PTS = 524288
NUM_LEVELS = 16
FEAT = 2
LOG2_T = 19
T = 1 << LOG2_T
TMASK = T - 1
MASK_RES = 128
R1 = MASK_RES + 1
PRIMES = (1, 2654435761, 805459861, 3674653429)
SCALE = 1.447
N_OUT = 16
N_FREQ = 6

NC = 2
NS = 16
NW = NC * NS

B = 128
NV = B // 16

_F32 = jnp.float32
_I32 = jnp.int32
_U32 = jnp.uint32


def _gen_hash_level(xv, resf, lofs_u, idx_buf, w_buf, ndim):
    ncor = 1 << ndim

    def vbody(v, carry):
        off = v * 16
        xs = [xv[pl.ds(d * B + off, 16)] for d in range(ndim)]
        pos = [x * resf for x in xs]
        pi = [p.astype(_I32) for p in pos]
        pf = [q.astype(_F32) for q in pi]
        fr = [p - q for p, q in zip(pos, pf)]
        g = [1.0 - f for f in fr]
        ha = []
        hb = []
        for d in range(ndim):
            u = pi[d].astype(_U32)
            a = u * _U32(PRIMES[d]) if PRIMES[d] != 1 else u
            ha.append(a)
            hb.append(a + _U32(PRIMES[d] & 0xFFFFFFFF))
        p01 = (g[0] * g[1], fr[0] * g[1], g[0] * fr[1], fr[0] * fr[1])
        if ndim == 4:
            p23 = (g[2] * g[3], fr[2] * g[3], g[2] * fr[3], fr[2] * fr[3])
        for corner in range(ncor):
            h = hb[0] if (corner & 1) else ha[0]
            for d in range(1, ndim):
                h = h ^ (hb[d] if ((corner >> d) & 1) else ha[d])
            idx = (h & _U32(TMASK)) + lofs_u
            idx_buf[pl.ds(corner * B + off, 16)] = idx.astype(_I32)
            if ndim == 4:
                w = p01[corner & 3] * p23[(corner >> 2) & 3]
            else:
                w = p01[corner & 3] * (fr[2] if (corner & 4) else g[2])
            w_buf[pl.ds(corner * B + off, 16)] = w
        return carry

    lax.fori_loop(0, NV, vbody, 0)


def _accum_level(g0, g1, w_buf, acc_ref, acc_off, ndim):
    ncor = 1 << ndim

    def vbody(v, carry):
        off = v * 16
        acc0 = jnp.zeros((16,), _F32)
        acc1 = jnp.zeros((16,), _F32)
        for c in range(ncor):
            w = w_buf[pl.ds(c * B + off, 16)]
            f0 = g0[pl.ds(c * B + off, 16)]
            f1 = g1[pl.ds(c * B + off, 16)]
            acc0 = acc0 + w * f0
            acc1 = acc1 + w * f1
        acc_ref[pl.ds(acc_off + off, 16)] = acc0
        acc_ref[pl.ds(acc_off + B + off, 16)] = acc1
        return carry

    lax.fori_loop(0, NV, vbody, 0)


def _gen_mask(xv, idx_buf, w_buf):
    def vbody(v, carry):
        off = v * 16
        xs = [xv[pl.ds(d * B + off, 16)] for d in range(3)]
        pos = [x * float(MASK_RES) for x in xs]
        pi = [p.astype(_I32) for p in pos]
        pf = [q.astype(_F32) for q in pi]
        fr = [p - q for p, q in zip(pos, pf)]
        g = [1.0 - f for f in fr]
        base = pi[0] + R1 * pi[1] + (R1 * R1) * pi[2]
        p01 = (g[0] * g[1], fr[0] * g[1], g[0] * fr[1], fr[0] * fr[1])
        for corner in range(8):
            o0, o1, o2 = corner & 1, (corner >> 1) & 1, (corner >> 2) & 1
            idx = base + (o0 + R1 * o1 + R1 * R1 * o2)
            idx_buf[pl.ds(corner * B + off, 16)] = idx
            w = p01[corner & 3] * (fr[2] if o2 else g[2])
            w_buf[pl.ds(corner * B + off, 16)] = w
        return carry

    lax.fori_loop(0, NV, vbody, 0)


def _sc_body(xT, st0, st1, tt0, tt1, mt, out,
             xv, idx_s, w_s, g_s0, g_s1, idx_t, w_t, g_t0, g_t1,
             idx_m, w_m, g_m, acc_s, acc_t, outbuf, sem_s, sem_t, sem_m):
    wid = lax.axis_index("s") * NC + lax.axis_index("c")
    ppw = N_PTS // NW
    nch = ppw // B
    wbase = wid * ppw
    iota16 = lax.iota(_I32, 16)

    def chunk_body(i, carry):
        gbase = wbase + i * B
        for d in range(4):
            pltpu.sync_copy(xT.at[d, pl.ds(gbase, B)], xv.at[pl.ds(d * B, B)])

        _gen_mask(xv, idx_m, w_m)
        mcps = []
        for c in range(8):
            mcps.append(pltpu.async_copy(
                mt.at[idx_m.at[pl.ds(c * B, B)]],
                g_m.at[pl.ds(c * B, B)], sem_m))

        def lev_body(l, resv):
            resf = resv.astype(_I32).astype(_F32)
            lofs_u = l.astype(_U32) * _U32(T)
            _gen_hash_level(xv, resf, lofs_u, idx_s, w_s, 3)
            scps = []
            for c in range(8):
                ids = idx_s.at[pl.ds(c * B, B)]
                scps.append(pltpu.async_copy(
                    st0.at[ids], g_s0.at[pl.ds(c * B, B)], sem_s))
                scps.append(pltpu.async_copy(
                    st1.at[ids], g_s1.at[pl.ds(c * B, B)], sem_s))
            _gen_hash_level(xv, resf, lofs_u, idx_t, w_t, 4)
            tcps = []
            for c in range(16):
                idt = idx_t.at[pl.ds(c * B, B)]
                tcps.append(pltpu.async_copy(
                    tt0.at[idt], g_t0.at[pl.ds(c * B, B)], sem_t))
                tcps.append(pltpu.async_copy(
                    tt1.at[idt], g_t1.at[pl.ds(c * B, B)], sem_t))
            for cp in scps:
                cp.wait()
            acc_off = (2 * l) * B
            _accum_level(g_s0, g_s1, w_s, acc_s, acc_off, 3)
            for cp in tcps:
                cp.wait()
            _accum_level(g_t0, g_t1, w_t, acc_t, acc_off, 4)
            return resv * _F32(SCALE)

        lax.fori_loop(0, NUM_LEVELS, lev_body, _F32(16.0))

        for cp in mcps:
            cp.wait()

        def blend_body(v, c2):
            off = v * 16
            macc = jnp.zeros((16,), _F32)
            for c in range(8):
                w = w_m[pl.ds(c * B + off, 16)]
                mv = g_m[pl.ds(c * B + off, 16)]
                macc = macc + w * mv
            m = 1.0 / (1.0 + jnp.exp(-macc))
            for p in range(32):
                s = acc_s[pl.ds(p * B + off, 16)]
                t = acc_t[pl.ds(p * B + off, 16)]
                bv = t + m * (s - t)
                outbuf[pl.ds(p * B + off, 16)] = bv
            return c2

        lax.fori_loop(0, NV, blend_body, 0)
        pltpu.sync_copy(outbuf, out.at[pl.ds(gbase * 32, B * 32)])
        return carry

    lax.fori_loop(0, nch, chunk_body, 0)


def _sc_blend(xT, st0, st1, tt0, tt1, mt):
    mesh = plsc.VectorSubcoreMesh(core_axis_name="c", subcore_axis_name="s")
    f = functools.partial(
        pl.kernel,
        mesh=mesh,
        out_type=jax.ShapeDtypeStruct((N_PTS * 32,), _F32),
        scratch_types=[
            pltpu.VMEM((4 * B,), _F32),
            pltpu.VMEM((8 * B,), _I32),
            pltpu.VMEM((8 * B,), _F32),
            pltpu.VMEM((8 * B,), _F32),
            pltpu.VMEM((8 * B,), _F32),
            pltpu.VMEM((16 * B,), _I32),
            pltpu.VMEM((16 * B,), _F32),
            pltpu.VMEM((16 * B,), _F32),
            pltpu.VMEM((16 * B,), _F32),
            pltpu.VMEM((8 * B,), _I32),
            pltpu.VMEM((8 * B,), _F32),
            pltpu.VMEM((8 * B,), _F32),
            pltpu.VMEM((32 * B,), _F32),
            pltpu.VMEM((32 * B,), _F32),
            pltpu.VMEM((32 * B,), _F32),
            pltpu.SemaphoreType.DMA,
            pltpu.SemaphoreType.DMA,
            pltpu.SemaphoreType.DMA,
        ],
    )(_sc_body)
    return f(xT, st0, st1, tt0, tt1, mt)


_TCG = 32


def _tc_head_body(x_ref, bl_ref, w_ref, b_ref, o_ref):
    t = x_ref[:, 3:4]
    acc = b_ref[...]
    for i in range(N_FREQ):
        f = _F32((2.0 ** i) * np.pi)
        acc = acc + jnp.sin(t * f) * w_ref[32 + 2 * i:33 + 2 * i, :]
        acc = acc + jnp.cos(t * f) * w_ref[33 + 2 * i:34 + 2 * i, :]
    w32 = w_ref[0:32, :]
    for g in range(_TCG):
        r = lax.dot_general(bl_ref[g], w32, (((0,), (0,)), ((), ())),
                            preferred_element_type=_F32,
                            precision=lax.Precision.HIGHEST)
        o_ref[pl.ds(g * B, B), :] = acc[g * B:(g + 1) * B] + r


def _tc_head(x, blended3, W, b):
    blkp = _TCG * B
    grid = (N_PTS // blkp,)
    return pl.pallas_call(
        _tc_head_body,
        grid=grid,
        in_specs=[
            pl.BlockSpec((blkp, 4), lambda i: (i, 0)),
            pl.BlockSpec((_TCG, 32, B), lambda i: (i, 0, 0)),
            pl.BlockSpec((44, N_OUT), lambda i: (0, 0)),
            pl.BlockSpec((1, N_OUT), lambda i: (0, 0)),
        ],
        out_specs=pl.BlockSpec((blkp, N_OUT), lambda i: (i, 0)),
        out_shape=jax.ShapeDtypeStruct((N_PTS, N_OUT), _F32),
    )(x, blended3, W, b)


@jax.jit
def kernel(x, spatial_table, temporal_table, mask_table, W, b):
    xT = x.T
    st0 = spatial_table[:, :, 0].reshape(-1)
    st1 = spatial_table[:, :, 1].reshape(-1)
    tt0 = temporal_table[:, :, 0].reshape(-1)
    tt1 = temporal_table[:, :, 1].reshape(-1)
    mt = mask_table.reshape(-1)
    blended3 = _sc_blend(xT, st0, st1, tt0, tt1, mt).reshape(
        N_PTS // B, 32, B)
    return _tc_head(x, blended3, W, b.reshape(1, N_OUT))


# speedup vs baseline: 1.6767x; 1.3933x over previous
import functools

import jax
import jax.numpy as jnp
import numpy as np
from jax import lax
from jax.experimental import pallas as pl
from jax.experimental.pallas import tpu as pltpu
from jax.experimental.pallas import tpu_sc as plsc

N_PTS = 524288
NUM_LEVELS = 16
FEAT = 2
LOG2_T = 19
T = 1 << LOG2_T
TMASK = T - 1
MASK_RES = 128
R1 = MASK_RES + 1
PRIMES = (1, 2654435761, 805459861, 3674653429)
SCALE = 1.447
N_OUT = 16
N_FREQ = 6

NC = 2
NS = 16
NW = NC * NS

B = 128
NV = B // 16

_F32 = jnp.float32
_I32 = jnp.int32
_U32 = jnp.uint32


def _gen_hash_level(xv, resf, lofs_u, idx_buf, w_buf, ndim):
    ncor = 1 << ndim

    def vbody(v, carry):
        off = v * 16
        xs = [xv[pl.ds(d * B + off, 16)] for d in range(ndim)]
        pos = [x * resf for x in xs]
        pi = [p.astype(_I32) for p in pos]
        pf = [q.astype(_F32) for q in pi]
        fr = [p - q for p, q in zip(pos, pf)]
        g = [1.0 - f for f in fr]
        ha = []
        hb = []
        for d in range(ndim):
            u = pi[d].astype(_U32)
            a = u * _U32(PRIMES[d]) if PRIMES[d] != 1 else u
            ha.append(a)
            hb.append(a + _U32(PRIMES[d] & 0xFFFFFFFF))
        p01 = (g[0] * g[1], fr[0] * g[1], g[0] * fr[1], fr[0] * fr[1])
        if ndim == 4:
            p23 = (g[2] * g[3], fr[2] * g[3], g[2] * fr[3], fr[2] * fr[3])
        for corner in range(ncor):
            h = hb[0] if (corner & 1) else ha[0]
            for d in range(1, ndim):
                h = h ^ (hb[d] if ((corner >> d) & 1) else ha[d])
            idx = (h & _U32(TMASK)) + lofs_u
            idx_buf[pl.ds(corner * B + off, 16)] = idx.astype(_I32)
            if ndim == 4:
                w = p01[corner & 3] * p23[(corner >> 2) & 3]
            else:
                w = p01[corner & 3] * (fr[2] if (corner & 4) else g[2])
            w_buf[pl.ds(corner * B + off, 16)] = w
        return carry

    lax.fori_loop(0, NV, vbody, 0)


def _accum_level(g_buf, w_buf, acc_ref, acc_off, ndim):
    ncor = 1 << ndim

    def vbody(v, carry):
        off = v * 16
        acc0 = jnp.zeros((16,), _F32)
        acc1 = jnp.zeros((16,), _F32)
        for c in range(ncor):
            w = w_buf[pl.ds(c * B + off, 16)]
            u = lax.bitcast_convert_type(g_buf[pl.ds(c * B + off, 16)],
                                         _U32)
            f0 = lax.bitcast_convert_type(u << 16, _F32)
            f1 = lax.bitcast_convert_type(u & _U32(0xFFFF0000), _F32)
            acc0 = acc0 + w * f0
            acc1 = acc1 + w * f1
        acc_ref[pl.ds(acc_off + off, 16)] = acc0
        acc_ref[pl.ds(acc_off + B + off, 16)] = acc1
        return carry

    lax.fori_loop(0, NV, vbody, 0)


def _gen_mask(xv, idx_buf, w_buf):
    def vbody(v, carry):
        off = v * 16
        xs = [xv[pl.ds(d * B + off, 16)] for d in range(3)]
        pos = [x * float(MASK_RES) for x in xs]
        pi = [p.astype(_I32) for p in pos]
        pf = [q.astype(_F32) for q in pi]
        fr = [p - q for p, q in zip(pos, pf)]
        g = [1.0 - f for f in fr]
        base = pi[0] + R1 * pi[1] + (R1 * R1) * pi[2]
        p01 = (g[0] * g[1], fr[0] * g[1], g[0] * fr[1], fr[0] * fr[1])
        for corner in range(8):
            o0, o1, o2 = corner & 1, (corner >> 1) & 1, (corner >> 2) & 1
            idx = base + (o0 + R1 * o1 + R1 * R1 * o2)
            idx_buf[pl.ds(corner * B + off, 16)] = idx
            w = p01[corner & 3] * (fr[2] if o2 else g[2])
            w_buf[pl.ds(corner * B + off, 16)] = w
        return carry

    lax.fori_loop(0, NV, vbody, 0)


def _sc_body(xT, st, tt, mt, out,
             xv, idx_s, w_s, g_s, idx_t, w_t, g_t,
             idx_m, w_m, g_m, acc_s, acc_t, outbuf, sem_s, sem_t, sem_m):
    wid = lax.axis_index("s") * NC + lax.axis_index("c")
    ppw = N_PTS // NW
    nch = ppw // B
    wbase = wid * ppw
    iota16 = lax.iota(_I32, 16)

    def chunk_body(i, carry):
        gbase = wbase + i * B
        for d in range(4):
            pltpu.sync_copy(xT.at[d, pl.ds(gbase, B)], xv.at[pl.ds(d * B, B)])

        _gen_mask(xv, idx_m, w_m)
        mcps = []
        for c in range(8):
            mcps.append(pltpu.async_copy(
                mt.at[idx_m.at[pl.ds(c * B, B)]],
                g_m.at[pl.ds(c * B, B)], sem_m))

        def lev_body(l, resv):
            resf = resv.astype(_I32).astype(_F32)
            lofs_u = l.astype(_U32) * _U32(T)
            _gen_hash_level(xv, resf, lofs_u, idx_s, w_s, 3)
            scps = []
            for c in range(8):
                scps.append(pltpu.async_copy(
                    st.at[idx_s.at[pl.ds(c * B, B)]],
                    g_s.at[pl.ds(c * B, B)], sem_s))
            _gen_hash_level(xv, resf, lofs_u, idx_t, w_t, 4)
            tcps = []
            for c in range(16):
                tcps.append(pltpu.async_copy(
                    tt.at[idx_t.at[pl.ds(c * B, B)]],
                    g_t.at[pl.ds(c * B, B)], sem_t))
            for cp in scps:
                cp.wait()
            acc_off = (2 * l) * B
            _accum_level(g_s, w_s, acc_s, acc_off, 3)
            for cp in tcps:
                cp.wait()
            _accum_level(g_t, w_t, acc_t, acc_off, 4)
            return resv * _F32(SCALE)

        lax.fori_loop(0, NUM_LEVELS, lev_body, _F32(16.0))

        for cp in mcps:
            cp.wait()

        def blend_body(v, c2):
            off = v * 16
            macc = jnp.zeros((16,), _F32)
            for c in range(8):
                w = w_m[pl.ds(c * B + off, 16)]
                mv = g_m[pl.ds(c * B + off, 16)]
                macc = macc + w * mv
            m = 1.0 / (1.0 + jnp.exp(-macc))
            for p in range(32):
                s = acc_s[pl.ds(p * B + off, 16)]
                t = acc_t[pl.ds(p * B + off, 16)]
                bv = t + m * (s - t)
                outbuf[pl.ds(p * B + off, 16)] = bv
            return c2

        lax.fori_loop(0, NV, blend_body, 0)
        pltpu.sync_copy(outbuf, out.at[pl.ds(gbase * 32, B * 32)])
        return carry

    lax.fori_loop(0, nch, chunk_body, 0)


def _sc_blend(xT, st, tt, mt):
    mesh = plsc.VectorSubcoreMesh(core_axis_name="c", subcore_axis_name="s")
    f = functools.partial(
        pl.kernel,
        mesh=mesh,
        out_type=jax.ShapeDtypeStruct((N_PTS * 32,), _F32),
        scratch_types=[
            pltpu.VMEM((4 * B,), _F32),
            pltpu.VMEM((8 * B,), _I32),
            pltpu.VMEM((8 * B,), _F32),
            pltpu.VMEM((8 * B,), _F32),
            pltpu.VMEM((16 * B,), _I32),
            pltpu.VMEM((16 * B,), _F32),
            pltpu.VMEM((16 * B,), _F32),
            pltpu.VMEM((8 * B,), _I32),
            pltpu.VMEM((8 * B,), _F32),
            pltpu.VMEM((8 * B,), _F32),
            pltpu.VMEM((32 * B,), _F32),
            pltpu.VMEM((32 * B,), _F32),
            pltpu.VMEM((32 * B,), _F32),
            pltpu.SemaphoreType.DMA,
            pltpu.SemaphoreType.DMA,
            pltpu.SemaphoreType.DMA,
        ],
    )(_sc_body)
    return f(xT, st, tt, mt)


_TCG = 32


def _tc_head_body(x_ref, bl_ref, w_ref, b_ref, o_ref):
    t = x_ref[:, 3:4]
    acc = b_ref[...]
    for i in range(N_FREQ):
        f = _F32((2.0 ** i) * np.pi)
        acc = acc + jnp.sin(t * f) * w_ref[32 + 2 * i:33 + 2 * i, :]
        acc = acc + jnp.cos(t * f) * w_ref[33 + 2 * i:34 + 2 * i, :]
    w32 = w_ref[0:32, :]
    for g in range(_TCG):
        r = lax.dot_general(bl_ref[g], w32, (((0,), (0,)), ((), ())),
                            preferred_element_type=_F32,
                            precision=lax.Precision.HIGHEST)
        o_ref[pl.ds(g * B, B), :] = acc[g * B:(g + 1) * B] + r


def _tc_head(x, blended3, W, b):
    blkp = _TCG * B
    grid = (N_PTS // blkp,)
    return pl.pallas_call(
        _tc_head_body,
        grid=grid,
        in_specs=[
            pl.BlockSpec((blkp, 4), lambda i: (i, 0)),
            pl.BlockSpec((_TCG, 32, B), lambda i: (i, 0, 0)),
            pl.BlockSpec((44, N_OUT), lambda i: (0, 0)),
            pl.BlockSpec((1, N_OUT), lambda i: (0, 0)),
        ],
        out_specs=pl.BlockSpec((blkp, N_OUT), lambda i: (i, 0)),
        out_shape=jax.ShapeDtypeStruct((N_PTS, N_OUT), _F32),
    )(x, blended3, W, b)


def _pack_table(t):
    tb = t.reshape(NUM_LEVELS * T, FEAT).astype(jnp.bfloat16)
    return lax.bitcast_convert_type(tb, _F32)


@jax.jit
def kernel(x, spatial_table, temporal_table, mask_table, W, b):
    xT = x.T
    st = _pack_table(spatial_table)
    tt = _pack_table(temporal_table)
    mt = mask_table.reshape(-1)
    blended3 = _sc_blend(xT, st, tt, mt).reshape(N_PTS // B, 32, B)
    return _tc_head(x, blended3, W, b.reshape(1, N_OUT))


# speedup vs baseline: 2.9997x; 1.7891x over previous
import functools

import jax
import jax.numpy as jnp
import numpy as np
from jax import lax
from jax.experimental import pallas as pl
from jax.experimental.pallas import tpu as pltpu
from jax.experimental.pallas import tpu_sc as plsc

N_PTS = 524288
NUM_LEVELS = 16
FEAT = 2
LOG2_T = 19
T = 1 << LOG2_T
TMASK = T - 1
MASK_RES = 128
R1 = MASK_RES + 1
PRIMES = (1, 2654435761, 805459861, 3674653429)
SCALE = 1.447
N_OUT = 16
N_FREQ = 6

NC = 2
NS = 16
NW = NC * NS

B = 128
NV = B // 16

_F32 = jnp.float32
_I32 = jnp.int32
_U32 = jnp.uint32


def _gen_hash_level(xv, resf, lofs_u, idx_buf, w_buf, ndim):
    ncor = 1 << ndim

    def vbody(v, carry):
        off = v * 16
        xs = [xv[pl.ds(d * B + off, 16)] for d in range(ndim)]
        pos = [x * resf for x in xs]
        pi = [p.astype(_I32) for p in pos]
        pf = [q.astype(_F32) for q in pi]
        fr = [p - q for p, q in zip(pos, pf)]
        g = [1.0 - f for f in fr]
        ha = []
        hb = []
        for d in range(ndim):
            u = pi[d].astype(_U32)
            a = u * _U32(PRIMES[d]) if PRIMES[d] != 1 else u
            ha.append(a)
            hb.append(a + _U32(PRIMES[d] & 0xFFFFFFFF))
        p01 = (g[0] * g[1], fr[0] * g[1], g[0] * fr[1], fr[0] * fr[1])
        if ndim == 4:
            p23 = (g[2] * g[3], fr[2] * g[3], g[2] * fr[3], fr[2] * fr[3])
        for corner in range(ncor):
            h = hb[0] if (corner & 1) else ha[0]
            for d in range(1, ndim):
                h = h ^ (hb[d] if ((corner >> d) & 1) else ha[d])
            idx = (h & _U32(TMASK)) + lofs_u
            idx_buf[pl.ds(corner * B + off, 16)] = idx.astype(_I32)
            if ndim == 4:
                w = p01[corner & 3] * p23[(corner >> 2) & 3]
            else:
                w = p01[corner & 3] * (fr[2] if (corner & 4) else g[2])
            w_buf[pl.ds(corner * B + off, 16)] = w
        return carry

    lax.fori_loop(0, NV, vbody, 0)


def _accum_level(g_buf, w_buf, acc_ref, acc_off, ndim):
    ncor = 1 << ndim

    def vbody(v, carry):
        off = v * 16
        acc0 = jnp.zeros((16,), _F32)
        acc1 = jnp.zeros((16,), _F32)
        for c in range(ncor):
            w = w_buf[pl.ds(c * B + off, 16)]
            u = lax.bitcast_convert_type(g_buf[pl.ds(c * B + off, 16)],
                                         _U32)
            f0 = lax.bitcast_convert_type(u << 16, _F32)
            f1 = lax.bitcast_convert_type(u & _U32(0xFFFF0000), _F32)
            acc0 = acc0 + w * f0
            acc1 = acc1 + w * f1
        acc_ref[pl.ds(acc_off + off, 16)] = acc0
        acc_ref[pl.ds(acc_off + B + off, 16)] = acc1
        return carry

    lax.fori_loop(0, NV, vbody, 0)


def _gen_mask(xv, idx_buf, w_buf):
    def vbody(v, carry):
        off = v * 16
        xs = [xv[pl.ds(d * B + off, 16)] for d in range(3)]
        pos = [x * float(MASK_RES) for x in xs]
        pi = [p.astype(_I32) for p in pos]
        pf = [q.astype(_F32) for q in pi]
        fr = [p - q for p, q in zip(pos, pf)]
        g = [1.0 - f for f in fr]
        base = pi[0] + R1 * pi[1] + (R1 * R1) * pi[2]
        p01 = (g[0] * g[1], fr[0] * g[1], g[0] * fr[1], fr[0] * fr[1])
        for corner in range(8):
            o0, o1, o2 = corner & 1, (corner >> 1) & 1, (corner >> 2) & 1
            idx = base + (o0 + R1 * o1 + R1 * R1 * o2)
            idx_buf[pl.ds(corner * B + off, 16)] = idx
            w = p01[corner & 3] * (fr[2] if o2 else g[2])
            w_buf[pl.ds(corner * B + off, 16)] = w
        return carry

    lax.fori_loop(0, NV, vbody, 0)


def _sc_body(xT, st, tt, mt, out,
             xv, idx_s, w_s, g_s, idx_t, w_t, g_t,
             idx_m, w_m, g_m, acc_s, acc_t, outbuf,
             sem_s, sem_t, sem_m, sem_o):
    wid = lax.axis_index("s") * NC + lax.axis_index("c")
    ppw = N_PTS // NW
    nch = ppw // B
    wbase = wid * ppw
    iota16 = lax.iota(_I32, 16)

    def chunk_body(i, carry):
        gbase = wbase + i * B
        for d in range(4):
            pltpu.sync_copy(xT.at[d, pl.ds(gbase, B)], xv.at[pl.ds(d * B, B)])

        _gen_mask(xv, idx_m, w_m)
        mcps = []
        for c in range(8):
            mcps.append(pltpu.async_copy(
                mt.at[idx_m.at[pl.ds(c * B, B)]],
                g_m.at[pl.ds(c * B, B)], sem_m))

        def lev_body(l, resv):
            resf = resv.astype(_I32).astype(_F32)
            lofs_u = l.astype(_U32) * _U32(T)
            _gen_hash_level(xv, resf, lofs_u, idx_s, w_s, 3)
            scps = []
            for c in range(8):
                scps.append(pltpu.async_copy(
                    st.at[idx_s.at[pl.ds(c * B, B)]],
                    g_s.at[pl.ds(c * B, B)], sem_s))
            _gen_hash_level(xv, resf, lofs_u, idx_t, w_t, 4)
            tcps = []
            for c in range(16):
                tcps.append(pltpu.async_copy(
                    tt.at[idx_t.at[pl.ds(c * B, B)]],
                    g_t.at[pl.ds(c * B, B)], sem_t))
            for cp in scps:
                cp.wait()
            acc_off = (2 * l) * B
            _accum_level(g_s, w_s, acc_s, acc_off, 3)
            for cp in tcps:
                cp.wait()
            _accum_level(g_t, w_t, acc_t, acc_off, 4)
            return resv * _F32(SCALE)

        lax.fori_loop(0, NUM_LEVELS, lev_body, _F32(16.0))

        for cp in mcps:
            cp.wait()

        def blend_body(v, c2):
            off = v * 16
            macc = jnp.zeros((16,), _F32)
            for c in range(8):
                w = w_m[pl.ds(c * B + off, 16)]
                mv = g_m[pl.ds(c * B + off, 16)]
                macc = macc + w * mv
            m = 1.0 / (1.0 + jnp.exp(-macc))
            for p in range(32):
                s = acc_s[pl.ds(p * B + off, 16)]
                t = acc_t[pl.ds(p * B + off, 16)]
                bv = t + m * (s - t)
                outbuf[pl.ds(p * B + off, 16)] = bv
            return c2

        lax.fori_loop(0, NV, blend_body, 0)
        ocps = []
        for p in range(32):
            ocps.append(pltpu.async_copy(
                outbuf.at[pl.ds(p * B, B)],
                out.at[p, pl.ds(gbase, B)], sem_o))
        for cp in ocps:
            cp.wait()
        return carry

    lax.fori_loop(0, nch, chunk_body, 0)


def _sc_blend(xT, st, tt, mt):
    mesh = plsc.VectorSubcoreMesh(core_axis_name="c", subcore_axis_name="s")
    f = functools.partial(
        pl.kernel,
        mesh=mesh,
        out_type=jax.ShapeDtypeStruct((32, N_PTS), _F32),
        scratch_types=[
            pltpu.VMEM((4 * B,), _F32),
            pltpu.VMEM((8 * B,), _I32),
            pltpu.VMEM((8 * B,), _F32),
            pltpu.VMEM((8 * B,), _F32),
            pltpu.VMEM((16 * B,), _I32),
            pltpu.VMEM((16 * B,), _F32),
            pltpu.VMEM((16 * B,), _F32),
            pltpu.VMEM((8 * B,), _I32),
            pltpu.VMEM((8 * B,), _F32),
            pltpu.VMEM((8 * B,), _F32),
            pltpu.VMEM((32 * B,), _F32),
            pltpu.VMEM((32 * B,), _F32),
            pltpu.VMEM((32 * B,), _F32),
            pltpu.SemaphoreType.DMA,
            pltpu.SemaphoreType.DMA,
            pltpu.SemaphoreType.DMA,
            pltpu.SemaphoreType.DMA,
        ],
    )(_sc_body)
    return f(xT, st, tt, mt)


_TCBLK = 16384


def _tc_head_body(xt_ref, bl_ref, w_ref, b_ref, o_ref):
    t = xt_ref[3:4, :]
    rows = [bl_ref[...]]
    for i in range(N_FREQ):
        f = _F32((2.0 ** i) * np.pi)
        rows.append(jnp.sin(t * f))
        rows.append(jnp.cos(t * f))
    feats = jnp.concatenate(rows, axis=0)
    r = lax.dot_general(feats, w_ref[...], (((0,), (0,)), ((), ())),
                        preferred_element_type=_F32,
                        precision=lax.Precision.HIGHEST)
    o_ref[...] = r + b_ref[...]


def _tc_head(xT, blended, W, b):
    grid = (N_PTS // _TCBLK,)
    return pl.pallas_call(
        _tc_head_body,
        grid=grid,
        in_specs=[
            pl.BlockSpec((4, _TCBLK), lambda i: (0, i)),
            pl.BlockSpec((32, _TCBLK), lambda i: (0, i)),
            pl.BlockSpec((44, N_OUT), lambda i: (0, 0)),
            pl.BlockSpec((1, N_OUT), lambda i: (0, 0)),
        ],
        out_specs=pl.BlockSpec((_TCBLK, N_OUT), lambda i: (i, 0)),
        out_shape=jax.ShapeDtypeStruct((N_PTS, N_OUT), _F32),
    )(xT, blended, W, b)


def _pack_table(t):
    tb = t.reshape(NUM_LEVELS * T, FEAT).astype(jnp.bfloat16)
    return lax.bitcast_convert_type(tb, _F32)


@jax.jit
def kernel(x, spatial_table, temporal_table, mask_table, W, b):
    xT = x.T
    st = _pack_table(spatial_table)
    tt = _pack_table(temporal_table)
    mt = mask_table.reshape(-1)
    blended = _sc_blend(xT, st, tt, mt)
    return _tc_head(xT, blended, W, b.reshape(1, N_OUT))


# speedup vs baseline: 3.2537x; 1.0847x over previous
import functools

import jax
import jax.numpy as jnp
import numpy as np
from jax import lax
from jax.experimental import pallas as pl
from jax.experimental.pallas import tpu as pltpu
from jax.experimental.pallas import tpu_sc as plsc

N_PTS = 524288
NUM_LEVELS = 16
FEAT = 2
LOG2_T = 19
T = 1 << LOG2_T
TMASK = T - 1
MASK_RES = 128
R1 = MASK_RES + 1
PRIMES = (1, 2654435761, 805459861, 3674653429)
SCALE = 1.447
N_OUT = 16
N_FREQ = 6

NC = 2
NS = 16
NW = NC * NS

B = 128
NV = B // 16

_F32 = jnp.float32
_I32 = jnp.int32
_U32 = jnp.uint32


def _gen_hash_level(xv, resf, lofs_u, idx_buf, w_buf, ndim, base):
    ncor = 1 << ndim

    def vbody(v, carry):
        off = v * 16
        goff = base + off
        xs = [xv[pl.ds(d * B + off, 16)] for d in range(ndim)]
        pos = [x * resf for x in xs]
        pi = [p.astype(_I32) for p in pos]
        pf = [q.astype(_F32) for q in pi]
        fr = [p - q for p, q in zip(pos, pf)]
        g = [1.0 - f for f in fr]
        ha = []
        hb = []
        for d in range(ndim):
            u = pi[d].astype(_U32)
            a = u * _U32(PRIMES[d]) if PRIMES[d] != 1 else u
            ha.append(a)
            hb.append(a + _U32(PRIMES[d] & 0xFFFFFFFF))
        p01 = (g[0] * g[1], fr[0] * g[1], g[0] * fr[1], fr[0] * fr[1])
        if ndim == 4:
            p23 = (g[2] * g[3], fr[2] * g[3], g[2] * fr[3], fr[2] * fr[3])
        for corner in range(ncor):
            h = hb[0] if (corner & 1) else ha[0]
            for d in range(1, ndim):
                h = h ^ (hb[d] if ((corner >> d) & 1) else ha[d])
            idx = (h & _U32(TMASK)) + lofs_u
            idx_buf[pl.ds(corner * B + goff, 16)] = idx.astype(_I32)
            if ndim == 4:
                w = p01[corner & 3] * p23[(corner >> 2) & 3]
            else:
                w = p01[corner & 3] * (fr[2] if (corner & 4) else g[2])
            w_buf[pl.ds(corner * B + goff, 16)] = w
        return carry

    lax.fori_loop(0, NV, vbody, 0)


def _accum_level(g_buf, w_buf, acc_ref, acc_off, ndim, base):
    ncor = 1 << ndim

    def vbody(v, carry):
        off = v * 16
        goff = base + off
        acc0 = jnp.zeros((16,), _F32)
        acc1 = jnp.zeros((16,), _F32)
        for c in range(ncor):
            w = w_buf[pl.ds(c * B + goff, 16)]
            u = lax.bitcast_convert_type(g_buf[pl.ds(c * B + goff, 16)],
                                         _U32)
            f0 = lax.bitcast_convert_type(u << 16, _F32)
            f1 = lax.bitcast_convert_type(u & _U32(0xFFFF0000), _F32)
            acc0 = acc0 + w * f0
            acc1 = acc1 + w * f1
        acc_ref[pl.ds(acc_off + off, 16)] = acc0
        acc_ref[pl.ds(acc_off + B + off, 16)] = acc1
        return carry

    lax.fori_loop(0, NV, vbody, 0)


def _gen_mask(xv, idx_buf, w_buf):
    def vbody(v, carry):
        off = v * 16
        xs = [xv[pl.ds(d * B + off, 16)] for d in range(3)]
        pos = [x * float(MASK_RES) for x in xs]
        pi = [p.astype(_I32) for p in pos]
        pf = [q.astype(_F32) for q in pi]
        fr = [p - q for p, q in zip(pos, pf)]
        g = [1.0 - f for f in fr]
        base = pi[0] + R1 * pi[1] + (R1 * R1) * pi[2]
        p01 = (g[0] * g[1], fr[0] * g[1], g[0] * fr[1], fr[0] * fr[1])
        for corner in range(8):
            o0, o1, o2 = corner & 1, (corner >> 1) & 1, (corner >> 2) & 1
            idx = base + (o0 + R1 * o1 + R1 * R1 * o2)
            idx_buf[pl.ds(corner * B + off, 16)] = idx
            w = p01[corner & 3] * (fr[2] if o2 else g[2])
            w_buf[pl.ds(corner * B + off, 16)] = w
        return carry

    lax.fori_loop(0, NV, vbody, 0)


def _sc_body(xT, st, tt, mt, out,
             xv, idx_s, w_s, g_s, idx_t, w_t, g_t,
             idx_m, w_m, g_m, acc_s, acc_t, outbuf,
             sem_sa, sem_sb, sem_ta, sem_tb, sem_m, sem_o):
    wid = lax.axis_index("s") * NC + lax.axis_index("c")
    ppw = N_PTS // NW
    nch = ppw // B
    wbase = wid * ppw

    def fire(table, idx_buf, g_buf, sem, ncor, base):
        for c in range(ncor):
            pltpu.async_copy(
                table.at[idx_buf.at[pl.ds(base + c * B, B)]],
                g_buf.at[pl.ds(base + c * B, B)], sem)

    def drain(table, g_buf, sem, ncor, base):
        pltpu.make_async_copy(
            table.at[pl.ds(0, ncor * B)],
            g_buf.at[pl.ds(base, ncor * B)], sem).wait()

    def gen_fire(resf, l, base, sem_s, sem_t):
        lofs_u = l.astype(_U32) * _U32(T)
        _gen_hash_level(xv, resf, lofs_u, idx_s, w_s, 3, base)
        fire(st, idx_s, g_s, sem_s, 8, base)
        _gen_hash_level(xv, resf, lofs_u, idx_t, w_t, 4, 2 * base)
        fire(tt, idx_t, g_t, sem_t, 16, 2 * base)

    def drain_acc(l, base, sem_s, sem_t):
        acc_off = (2 * l) * B
        drain(st, g_s, sem_s, 8, base)
        _accum_level(g_s, w_s, acc_s, acc_off, 3, base)
        drain(tt, g_t, sem_t, 16, 2 * base)
        _accum_level(g_t, w_t, acc_t, acc_off, 4, 2 * base)

    HS = 8 * B

    def chunk_body(i, carry):
        gbase = wbase + i * B
        for d in range(4):
            pltpu.sync_copy(xT.at[d, pl.ds(gbase, B)], xv.at[pl.ds(d * B, B)])

        _gen_mask(xv, idx_m, w_m)
        mcps = []
        for c in range(8):
            mcps.append(pltpu.async_copy(
                mt.at[idx_m.at[pl.ds(c * B, B)]],
                g_m.at[pl.ds(c * B, B)], sem_m))

        gen_fire(_F32(16.0), jnp.int32(0), 0, sem_sa, sem_ta)

        def lev2_body(j, resv):
            l0 = 2 * j
            r1v = resv * _F32(SCALE)
            r2v = r1v * _F32(SCALE)
            resf1 = r1v.astype(_I32).astype(_F32)
            resf2 = r2v.astype(_I32).astype(_F32)
            gen_fire(resf1, l0 + 1, HS, sem_sb, sem_tb)
            drain_acc(l0, 0, sem_sa, sem_ta)

            @pl.when(j < (NUM_LEVELS // 2 - 1))
            def _():
                gen_fire(resf2, l0 + 2, 0, sem_sa, sem_ta)

            drain_acc(l0 + 1, HS, sem_sb, sem_tb)
            return r2v

        lax.fori_loop(0, NUM_LEVELS // 2, lev2_body, _F32(16.0))

        for cp in mcps:
            cp.wait()

        def blend_body(v, c2):
            off = v * 16
            macc = jnp.zeros((16,), _F32)
            for c in range(8):
                w = w_m[pl.ds(c * B + off, 16)]
                mv = g_m[pl.ds(c * B + off, 16)]
                macc = macc + w * mv
            m = 1.0 / (1.0 + jnp.exp(-macc))
            for p in range(32):
                s = acc_s[pl.ds(p * B + off, 16)]
                t = acc_t[pl.ds(p * B + off, 16)]
                bv = t + m * (s - t)
                outbuf[pl.ds(p * B + off, 16)] = bv
            return c2

        lax.fori_loop(0, NV, blend_body, 0)
        ocps = []
        for p in range(32):
            ocps.append(pltpu.async_copy(
                outbuf.at[pl.ds(p * B, B)],
                out.at[p, pl.ds(gbase, B)], sem_o))
        for cp in ocps:
            cp.wait()
        return carry

    lax.fori_loop(0, nch, chunk_body, 0)


def _sc_blend(xT, st, tt, mt):
    mesh = plsc.VectorSubcoreMesh(core_axis_name="c", subcore_axis_name="s")
    f = functools.partial(
        pl.kernel,
        mesh=mesh,
        out_type=jax.ShapeDtypeStruct((32, N_PTS), _F32),
        scratch_types=[
            pltpu.VMEM((4 * B,), _F32),
            pltpu.VMEM((16 * B,), _I32),
            pltpu.VMEM((16 * B,), _F32),
            pltpu.VMEM((16 * B,), _F32),
            pltpu.VMEM((32 * B,), _I32),
            pltpu.VMEM((32 * B,), _F32),
            pltpu.VMEM((32 * B,), _F32),
            pltpu.VMEM((8 * B,), _I32),
            pltpu.VMEM((8 * B,), _F32),
            pltpu.VMEM((8 * B,), _F32),
            pltpu.VMEM((32 * B,), _F32),
            pltpu.VMEM((32 * B,), _F32),
            pltpu.VMEM((32 * B,), _F32),
            pltpu.SemaphoreType.DMA,
            pltpu.SemaphoreType.DMA,
            pltpu.SemaphoreType.DMA,
            pltpu.SemaphoreType.DMA,
            pltpu.SemaphoreType.DMA,
            pltpu.SemaphoreType.DMA,
        ],
    )(_sc_body)
    return f(xT, st, tt, mt)


_TCBLK = 16384


def _tc_head_body(xt_ref, bl_ref, w_ref, b_ref, o_ref):
    t = xt_ref[3:4, :]
    rows = [bl_ref[...]]
    for i in range(N_FREQ):
        f = _F32((2.0 ** i) * np.pi)
        rows.append(jnp.sin(t * f))
        rows.append(jnp.cos(t * f))
    feats = jnp.concatenate(rows, axis=0)
    r = lax.dot_general(feats, w_ref[...], (((0,), (0,)), ((), ())),
                        preferred_element_type=_F32,
                        precision=lax.Precision.HIGHEST)
    o_ref[...] = r + b_ref[...]


def _tc_head(xT, blended, W, b):
    grid = (N_PTS // _TCBLK,)
    return pl.pallas_call(
        _tc_head_body,
        grid=grid,
        in_specs=[
            pl.BlockSpec((4, _TCBLK), lambda i: (0, i)),
            pl.BlockSpec((32, _TCBLK), lambda i: (0, i)),
            pl.BlockSpec((44, N_OUT), lambda i: (0, 0)),
            pl.BlockSpec((1, N_OUT), lambda i: (0, 0)),
        ],
        out_specs=pl.BlockSpec((_TCBLK, N_OUT), lambda i: (i, 0)),
        out_shape=jax.ShapeDtypeStruct((N_PTS, N_OUT), _F32),
    )(xT, blended, W, b)


def _pack_table(t):
    tb = t.reshape(NUM_LEVELS * T, FEAT).astype(jnp.bfloat16)
    return lax.bitcast_convert_type(tb, _F32)


@jax.jit
def kernel(x, spatial_table, temporal_table, mask_table, W, b):
    xT = x.T
    st = _pack_table(spatial_table)
    tt = _pack_table(temporal_table)
    mt = mask_table.reshape(-1)
    blended = _sc_blend(xT, st, tt, mt)
    return _tc_head(xT, blended, W, b.reshape(1, N_OUT))


# speedup vs baseline: 3.2562x; 1.0007x over previous
import functools

import jax
import jax.numpy as jnp
import numpy as np
from jax import lax
from jax.experimental import pallas as pl
from jax.experimental.pallas import tpu as pltpu
from jax.experimental.pallas import tpu_sc as plsc

N_PTS = 524288
NUM_LEVELS = 16
FEAT = 2
LOG2_T = 19
T = 1 << LOG2_T
TMASK = T - 1
MASK_RES = 128
R1 = MASK_RES + 1
PRIMES = (1, 2654435761, 805459861, 3674653429)
SCALE = 1.447
N_OUT = 16
N_FREQ = 6

NC = 2
NS = 16
NW = NC * NS

B = 128
NV = B // 16

_F32 = jnp.float32
_I32 = jnp.int32
_U32 = jnp.uint32


def _gen_hash_level(xv, resf, lofs_u, idx_buf, w_buf, ndim, base):
    ncor = 1 << ndim

    @plsc.parallel_loop(0, NV, unroll=2)
    def vbody(v):
        off = v * 16
        goff = base + off
        xs = [xv[pl.ds(d * B + off, 16)] for d in range(ndim)]
        pos = [x * resf for x in xs]
        pi = [p.astype(_I32) for p in pos]
        pf = [q.astype(_F32) for q in pi]
        fr = [p - q for p, q in zip(pos, pf)]
        g = [1.0 - f for f in fr]
        ha = []
        hb = []
        for d in range(ndim):
            u = pi[d].astype(_U32)
            a = u * _U32(PRIMES[d]) if PRIMES[d] != 1 else u
            ha.append(a)
            hb.append(a + _U32(PRIMES[d] & 0xFFFFFFFF))
        p01 = (g[0] * g[1], fr[0] * g[1], g[0] * fr[1], fr[0] * fr[1])
        if ndim == 4:
            p23 = (g[2] * g[3], fr[2] * g[3], g[2] * fr[3], fr[2] * fr[3])
        for corner in range(ncor):
            h = hb[0] if (corner & 1) else ha[0]
            for d in range(1, ndim):
                h = h ^ (hb[d] if ((corner >> d) & 1) else ha[d])
            idx = (h & _U32(TMASK)) + lofs_u
            idx_buf[pl.ds(corner * B + goff, 16)] = idx.astype(_I32)
            if ndim == 4:
                w = p01[corner & 3] * p23[(corner >> 2) & 3]
            else:
                w = p01[corner & 3] * (fr[2] if (corner & 4) else g[2])
            w_buf[pl.ds(corner * B + goff, 16)] = w


def _accum_level(g_buf, w_buf, acc_ref, acc_off, ndim, base):
    ncor = 1 << ndim

    @plsc.parallel_loop(0, NV, unroll=2)
    def vbody(v):
        off = v * 16
        goff = base + off
        acc0 = jnp.zeros((16,), _F32)
        acc1 = jnp.zeros((16,), _F32)
        for c in range(ncor):
            w = w_buf[pl.ds(c * B + goff, 16)]
            u = lax.bitcast_convert_type(g_buf[pl.ds(c * B + goff, 16)],
                                         _U32)
            f0 = lax.bitcast_convert_type(u << 16, _F32)
            f1 = lax.bitcast_convert_type(u & _U32(0xFFFF0000), _F32)
            acc0 = acc0 + w * f0
            acc1 = acc1 + w * f1
        acc_ref[pl.ds(acc_off + off, 16)] = acc0
        acc_ref[pl.ds(acc_off + B + off, 16)] = acc1


def _gen_mask(xv, idx_buf, w_buf):
    @plsc.parallel_loop(0, NV, unroll=2)
    def vbody(v):
        off = v * 16
        xs = [xv[pl.ds(d * B + off, 16)] for d in range(3)]
        pos = [x * float(MASK_RES) for x in xs]
        pi = [p.astype(_I32) for p in pos]
        pf = [q.astype(_F32) for q in pi]
        fr = [p - q for p, q in zip(pos, pf)]
        g = [1.0 - f for f in fr]
        base = pi[0] + R1 * pi[1] + (R1 * R1) * pi[2]
        p01 = (g[0] * g[1], fr[0] * g[1], g[0] * fr[1], fr[0] * fr[1])
        for corner in range(8):
            o0, o1, o2 = corner & 1, (corner >> 1) & 1, (corner >> 2) & 1
            idx = base + (o0 + R1 * o1 + R1 * R1 * o2)
            idx_buf[pl.ds(corner * B + off, 16)] = idx
            w = p01[corner & 3] * (fr[2] if o2 else g[2])
            w_buf[pl.ds(corner * B + off, 16)] = w


def _sc_body(xT, st, tt, mt, out,
             xv, idx_s, w_s, g_s, idx_t, w_t, g_t,
             idx_m, w_m, g_m, acc_s, acc_t, outbuf,
             sem_sa, sem_sb, sem_ta, sem_tb, sem_m, sem_o):
    wid = lax.axis_index("s") * NC + lax.axis_index("c")
    ppw = N_PTS // NW
    nch = ppw // B
    wbase = wid * ppw

    def fire(table, idx_buf, g_buf, sem, ncor, base):
        for c in range(ncor):
            pltpu.async_copy(
                table.at[idx_buf.at[pl.ds(base + c * B, B)]],
                g_buf.at[pl.ds(base + c * B, B)], sem)

    def drain(table, g_buf, sem, ncor, base):
        pltpu.make_async_copy(
            table.at[pl.ds(0, ncor * B)],
            g_buf.at[pl.ds(base, ncor * B)], sem).wait()

    def gen_fire(resf, l, base, sem_s, sem_t):
        lofs_u = l.astype(_U32) * _U32(T)
        _gen_hash_level(xv, resf, lofs_u, idx_s, w_s, 3, base)
        fire(st, idx_s, g_s, sem_s, 8, base)
        _gen_hash_level(xv, resf, lofs_u, idx_t, w_t, 4, 2 * base)
        fire(tt, idx_t, g_t, sem_t, 16, 2 * base)

    def drain_acc(l, base, sem_s, sem_t):
        acc_off = (2 * l) * B
        drain(st, g_s, sem_s, 8, base)
        _accum_level(g_s, w_s, acc_s, acc_off, 3, base)
        drain(tt, g_t, sem_t, 16, 2 * base)
        _accum_level(g_t, w_t, acc_t, acc_off, 4, 2 * base)

    HS = 8 * B

    def chunk_body(i, carry):
        gbase = wbase + i * B
        for d in range(4):
            pltpu.sync_copy(xT.at[d, pl.ds(gbase, B)], xv.at[pl.ds(d * B, B)])

        _gen_mask(xv, idx_m, w_m)
        mcps = []
        for c in range(8):
            mcps.append(pltpu.async_copy(
                mt.at[idx_m.at[pl.ds(c * B, B)]],
                g_m.at[pl.ds(c * B, B)], sem_m))

        gen_fire(_F32(16.0), jnp.int32(0), 0, sem_sa, sem_ta)

        def lev2_body(j, resv):
            l0 = 2 * j
            r1v = resv * _F32(SCALE)
            r2v = r1v * _F32(SCALE)
            resf1 = r1v.astype(_I32).astype(_F32)
            resf2 = r2v.astype(_I32).astype(_F32)
            gen_fire(resf1, l0 + 1, HS, sem_sb, sem_tb)
            drain_acc(l0, 0, sem_sa, sem_ta)

            @pl.when(j < (NUM_LEVELS // 2 - 1))
            def _():
                gen_fire(resf2, l0 + 2, 0, sem_sa, sem_ta)

            drain_acc(l0 + 1, HS, sem_sb, sem_tb)
            return r2v

        lax.fori_loop(0, NUM_LEVELS // 2, lev2_body, _F32(16.0))

        for cp in mcps:
            cp.wait()

        @plsc.parallel_loop(0, NV, unroll=2)
        def blend_body(v):
            off = v * 16
            macc = jnp.zeros((16,), _F32)
            for c in range(8):
                w = w_m[pl.ds(c * B + off, 16)]
                mv = g_m[pl.ds(c * B + off, 16)]
                macc = macc + w * mv
            m = 1.0 / (1.0 + jnp.exp(-macc))
            for p in range(32):
                s = acc_s[pl.ds(p * B + off, 16)]
                t = acc_t[pl.ds(p * B + off, 16)]
                bv = t + m * (s - t)
                outbuf[pl.ds(p * B + off, 16)] = bv
        ocps = []
        for p in range(32):
            ocps.append(pltpu.async_copy(
                outbuf.at[pl.ds(p * B, B)],
                out.at[p, pl.ds(gbase, B)], sem_o))
        for cp in ocps:
            cp.wait()
        return carry

    lax.fori_loop(0, nch, chunk_body, 0)


def _sc_blend(xT, st, tt, mt):
    mesh = plsc.VectorSubcoreMesh(core_axis_name="c", subcore_axis_name="s")
    f = functools.partial(
        pl.kernel,
        mesh=mesh,
        out_type=jax.ShapeDtypeStruct((32, N_PTS), _F32),
        scratch_types=[
            pltpu.VMEM((4 * B,), _F32),
            pltpu.VMEM((16 * B,), _I32),
            pltpu.VMEM((16 * B,), _F32),
            pltpu.VMEM((16 * B,), _F32),
            pltpu.VMEM((32 * B,), _I32),
            pltpu.VMEM((32 * B,), _F32),
            pltpu.VMEM((32 * B,), _F32),
            pltpu.VMEM((8 * B,), _I32),
            pltpu.VMEM((8 * B,), _F32),
            pltpu.VMEM((8 * B,), _F32),
            pltpu.VMEM((32 * B,), _F32),
            pltpu.VMEM((32 * B,), _F32),
            pltpu.VMEM((32 * B,), _F32),
            pltpu.SemaphoreType.DMA,
            pltpu.SemaphoreType.DMA,
            pltpu.SemaphoreType.DMA,
            pltpu.SemaphoreType.DMA,
            pltpu.SemaphoreType.DMA,
            pltpu.SemaphoreType.DMA,
        ],
    )(_sc_body)
    return f(xT, st, tt, mt)


_TCBLK = 16384


def _tc_head_body(xt_ref, bl_ref, w_ref, b_ref, o_ref):
    t = xt_ref[3:4, :]
    rows = [bl_ref[...]]
    for i in range(N_FREQ):
        f = _F32((2.0 ** i) * np.pi)
        rows.append(jnp.sin(t * f))
        rows.append(jnp.cos(t * f))
    feats = jnp.concatenate(rows, axis=0)
    r = lax.dot_general(feats, w_ref[...], (((0,), (0,)), ((), ())),
                        preferred_element_type=_F32,
                        precision=lax.Precision.HIGHEST)
    o_ref[...] = r + b_ref[...]


def _tc_head(xT, blended, W, b):
    grid = (N_PTS // _TCBLK,)
    return pl.pallas_call(
        _tc_head_body,
        grid=grid,
        in_specs=[
            pl.BlockSpec((4, _TCBLK), lambda i: (0, i)),
            pl.BlockSpec((32, _TCBLK), lambda i: (0, i)),
            pl.BlockSpec((44, N_OUT), lambda i: (0, 0)),
            pl.BlockSpec((1, N_OUT), lambda i: (0, 0)),
        ],
        out_specs=pl.BlockSpec((_TCBLK, N_OUT), lambda i: (i, 0)),
        out_shape=jax.ShapeDtypeStruct((N_PTS, N_OUT), _F32),
    )(xT, blended, W, b)


def _pack_table(t):
    tb = t.reshape(NUM_LEVELS * T, FEAT).astype(jnp.bfloat16)
    return lax.bitcast_convert_type(tb, _F32)


@jax.jit
def kernel(x, spatial_table, temporal_table, mask_table, W, b):
    xT = x.T
    st = _pack_table(spatial_table)
    tt = _pack_table(temporal_table)
    mt = mask_table.reshape(-1)
    blended = _sc_blend(xT, st, tt, mt)
    return _tc_head(xT, blended, W, b.reshape(1, N_OUT))
